# baseline (speedup 1.0000x reference)
import functools

import jax
import jax.numpy as jnp
from jax import lax
from jax.experimental import pallas as pl
from jax.experimental.pallas import tpu as pltpu
from jax.experimental.pallas import tpu_sc as plsc

N = 10000
D_IN = 128
D_HID = 128
D_OUT = 64
E = 320000

NC = 2
NS = 16
NW = NC * NS

CH = 128
E_PAD = 327680
EPT_D = E_PAD // NW
NCH_D = EPT_D // CH
EPT_E = E_PAD // NS
NCH_E = EPT_E // CH
N_PAD = 10240
DUMMY = N

STRIPE = N_PAD // NS

_mesh = plsc.VectorSubcoreMesh(
    core_axis_name="c", subcore_axis_name="s", num_cores=NC, num_subcores=NS)

_sc_params = pltpu.CompilerParams(
    needs_layout_passes=False, use_tc_tiling_on_sc=False)


def _deg_body(dst_hbm, out_hbm, dst_v, deg_v):
  c = lax.axis_index("c")
  s = lax.axis_index("s")
  tile = c * NS + s

  def zero(i, _):
    deg_v[pl.ds(i * 16, 16)] = jnp.zeros((16,), jnp.float32)
    return 0
  lax.fori_loop(0, N_PAD // 16, zero, 0)

  pltpu.sync_copy(dst_hbm.at[tile], dst_v)

  ones = jnp.ones((16,), jnp.float32)

  def count(j, _):
    for k in range(CH // 16):
      idx = dst_v[j, pl.ds(k * 16, 16)]
      plsc.addupdate_scatter(deg_v, [idx], ones)
    return 0
  lax.fori_loop(0, NCH_D, count, 0)

  pltpu.sync_copy(deg_v, out_hbm.at[tile])


def _deg_kernel(dst3):
  return pl.kernel(
      _deg_body,
      out_type=jax.ShapeDtypeStruct((NW, N_PAD), jnp.float32),
      mesh=_mesh,
      scratch_types=[
          pltpu.VMEM((NCH_D, CH), jnp.int32),
          pltpu.VMEM((N_PAD,), jnp.float32),
      ],
      compiler_params=_sc_params,
  )(dst3)


def _edge_body(h_hbm, idx_hbm, zeros_hbm, out_hbm,
               idx_a, idx_b, rows_a, rows_b, h_sh, acc_sh,
               sem_a, sem_b, sem_ia, sem_ib, *, dh):
  c = lax.axis_index("c")
  s = lax.axis_index("s")

  pltpu.sync_copy(h_hbm.at[c, pl.ds(s * STRIPE, STRIPE)],
                  h_sh.at[pl.ds(s * STRIPE, STRIPE)])
  pltpu.sync_copy(zeros_hbm.at[pl.ds(s * STRIPE, STRIPE)],
                  acc_sh.at[pl.ds(s * STRIPE, STRIPE)])
  plsc.subcore_barrier()

  pltpu.async_copy(idx_hbm.at[s, 0], idx_a, sem_ia)
  pltpu.async_copy(idx_hbm.at[s, 1], idx_b, sem_ib)
  pltpu.make_async_copy(idx_hbm.at[s, 0], idx_a, sem_ia).wait()
  pltpu.async_copy(h_sh.at[idx_a.at[0]], rows_a, sem_a)

  def half_step(j, idx_cur, idx_nxt, rows_cur, rows_nxt,
                s_cur, s_nxt, si_cur, si_nxt):
    @pl.when(j + 1 < NCH_E)
    def _():
      pltpu.make_async_copy(idx_hbm.at[s, j + 1], idx_nxt, si_nxt).wait()
      pltpu.async_copy(h_sh.at[idx_nxt.at[0]], rows_nxt, s_nxt)

    pltpu.make_async_copy(h_sh.at[idx_cur.at[0]], rows_cur, s_cur).wait()
    pltpu.sync_copy(rows_cur, acc_sh.at[idx_cur.at[1]], add=True)

    @pl.when(j + 2 < NCH_E)
    def _():
      pltpu.async_copy(idx_hbm.at[s, j + 2], idx_cur, si_cur)

  def step(j, _):
    even = lax.rem(j, 2) == 0

    @pl.when(even)
    def _():
      half_step(j, idx_a, idx_b, rows_a, rows_b,
                sem_a, sem_b, sem_ia, sem_ib)

    @pl.when(jnp.logical_not(even))
    def _():
      half_step(j, idx_b, idx_a, rows_b, rows_a,
                sem_b, sem_a, sem_ib, sem_ia)

    return 0

  lax.fori_loop(0, NCH_E, step, 0)
  plsc.subcore_barrier()

  pltpu.sync_copy(acc_sh.at[pl.ds(s * STRIPE, STRIPE)],
                  out_hbm.at[c, pl.ds(s * STRIPE, STRIPE)])


def _edge_kernel(h, idx4, zeros_nd, dh):
  body = functools.partial(_edge_body, dh=dh)
  return pl.kernel(
      body,
      out_type=jax.ShapeDtypeStruct((NC, N_PAD, dh), jnp.float32),
      mesh=_mesh,
      scratch_types=[
          pltpu.VMEM((2, CH), jnp.int32),
          pltpu.VMEM((2, CH), jnp.int32),
          pltpu.VMEM((CH, dh), jnp.float32),
          pltpu.VMEM((CH, dh), jnp.float32),
          pltpu.VMEM_SHARED((N_PAD, dh), jnp.float32),
          pltpu.VMEM_SHARED((N_PAD, dh), jnp.float32),
          pltpu.SemaphoreType.DMA,
          pltpu.SemaphoreType.DMA,
          pltpu.SemaphoreType.DMA,
          pltpu.SemaphoreType.DMA,
      ],
      compiler_params=_sc_params,
  )(h, idx4, zeros_nd)


_R = 2000


def _dis(degt_ref):
  deg = 1.0 + jnp.sum(degt_ref[...], axis=1, keepdims=True)
  return lax.rsqrt(deg)


def _scale_in_body(x_ref, w_ref, degt_ref, out_ref):
  dis = _dis(degt_ref)
  h = jnp.dot(x_ref[...], w_ref[0], preferred_element_type=jnp.float32)
  out_ref[0] = h * dis


def _tc_scale_in(x, w, degt, d_in, dh):
  return pl.pallas_call(
      _scale_in_body,
      grid=(N // _R, NC),
      in_specs=[
          pl.BlockSpec((_R, d_in), lambda j, c: (j, 0)),
          pl.BlockSpec((1, d_in, dh), lambda j, c: (c, 0, 0)),
          pl.BlockSpec((_R, NW), lambda j, c: (j, 0)),
      ],
      out_specs=pl.BlockSpec((1, _R, dh), lambda j, c: (c, j, 0)),
      out_shape=jax.ShapeDtypeStruct((NC, N_PAD, dh), jnp.float32),
  )(x, w, degt)


def _mid_body(acc_ref, hp_ref, degt_ref, b_ref, w_ref, out_ref):
  dis = _dis(degt_ref)
  tot = jnp.concatenate(
      [acc_ref[0] + hp_ref[0], acc_ref[1] + hp_ref[1]], axis=1)
  z = jnp.maximum(dis * tot + b_ref[...], 0.0)
  h2 = jnp.dot(z, w_ref[0], preferred_element_type=jnp.float32)
  out_ref[0] = h2 * dis


def _tc_mid(acc, hp, degt, b, w, dh_in, dh_out):
  return pl.pallas_call(
      _mid_body,
      grid=(N // _R, NC),
      in_specs=[
          pl.BlockSpec((NC, _R, dh_in), lambda j, c: (0, j, 0)),
          pl.BlockSpec((NC, _R, dh_in), lambda j, c: (0, j, 0)),
          pl.BlockSpec((_R, NW), lambda j, c: (j, 0)),
          pl.BlockSpec((1, 2 * dh_in), lambda j, c: (0, 0)),
          pl.BlockSpec((1, 2 * dh_in, dh_out), lambda j, c: (c, 0, 0)),
      ],
      out_specs=pl.BlockSpec((1, _R, dh_out), lambda j, c: (c, j, 0)),
      out_shape=jax.ShapeDtypeStruct((NC, N_PAD, dh_out), jnp.float32),
  )(acc, hp, degt, b, w)


def _final_body(acc_ref, hp_ref, degt_ref, b_ref, out_ref):
  dis = _dis(degt_ref)
  tot = jnp.concatenate(
      [acc_ref[0] + hp_ref[0], acc_ref[1] + hp_ref[1]], axis=1)
  out_ref[...] = jnp.maximum(dis * tot + b_ref[...], 0.0)


def _tc_final(acc, hp, degt, b, dh):
  return pl.pallas_call(
      _final_body,
      grid=(N // _R,),
      in_specs=[
          pl.BlockSpec((NC, _R, dh), lambda j: (0, j, 0)),
          pl.BlockSpec((NC, _R, dh), lambda j: (0, j, 0)),
          pl.BlockSpec((_R, NW), lambda j: (j, 0)),
          pl.BlockSpec((1, 2 * dh), lambda j: (0, 0)),
      ],
      out_specs=pl.BlockSpec((_R, 2 * dh), lambda j: (j, 0)),
      out_shape=jax.ShapeDtypeStruct((N, 2 * dh), jnp.float32),
  )(acc, hp, degt, b)


def kernel(x, edge_index, W1, b1, W2, b2):
  src = edge_index[0].astype(jnp.int32)
  dst = edge_index[1].astype(jnp.int32)
  pad = E_PAD - E
  src_p = jnp.concatenate([src, jnp.zeros((pad,), jnp.int32)])
  dst_p = jnp.concatenate([dst, jnp.full((pad,), DUMMY, jnp.int32)])
  dst3 = dst_p.reshape(NW, NCH_D, CH)
  idx4 = jnp.stack([src_p.reshape(NS, NCH_E, CH),
                    dst_p.reshape(NS, NCH_E, CH)], axis=2)

  deg_parts = _deg_kernel(dst3)
  degt = deg_parts.T[:N]

  zeros_64 = jnp.zeros((N_PAD, D_HID // 2), jnp.float32)
  zeros_32 = jnp.zeros((N_PAD, D_OUT // 2), jnp.float32)

  W1s = W1.reshape(D_IN, NC, D_HID // 2).transpose(1, 0, 2)
  W2s = W2.reshape(D_HID, NC, D_OUT // 2).transpose(1, 0, 2)

  h1p = _tc_scale_in(x, W1s, degt, D_IN, D_HID // 2)
  acc1 = _edge_kernel(h1p, idx4, zeros_64, D_HID // 2)
  h2p = _tc_mid(acc1, h1p, degt, b1.reshape(1, D_HID), W2s,
                D_HID // 2, D_OUT // 2)
  acc2 = _edge_kernel(h2p, idx4, zeros_32, D_OUT // 2)
  out = _tc_final(acc2, h2p, degt, b2.reshape(1, D_OUT), D_OUT // 2)
  return out


# speedup vs baseline: 1.0030x; 1.0030x over previous
import functools

import jax
import jax.numpy as jnp
from jax import lax
from jax.experimental import pallas as pl
from jax.experimental.pallas import tpu as pltpu
from jax.experimental.pallas import tpu_sc as plsc

N = 10000
D_IN = 128
D_HID = 128
D_OUT = 64
E = 320000

NC = 2
NS = 16
NW = NC * NS

CH = 128
E_PAD = 327680
EPT_D = E_PAD // NW
NCH_D = EPT_D // CH
EPT_E = E_PAD // NS
NCH_E = EPT_E // CH
N_PAD = 10240
DUMMY = N

STRIPE = N_PAD // NS

_mesh = plsc.VectorSubcoreMesh(
    core_axis_name="c", subcore_axis_name="s", num_cores=NC, num_subcores=NS)

_sc_params = pltpu.CompilerParams(
    needs_layout_passes=False, use_tc_tiling_on_sc=False)


def _deg_body(dst_hbm, out_hbm, dst_v, deg_v):
  c = lax.axis_index("c")
  s = lax.axis_index("s")
  tile = c * NS + s

  def zero(i, _):
    deg_v[pl.ds(i * 16, 16)] = jnp.zeros((16,), jnp.float32)
    return 0
  lax.fori_loop(0, N_PAD // 16, zero, 0)

  pltpu.sync_copy(dst_hbm.at[tile], dst_v)

  ones = jnp.ones((16,), jnp.float32)

  def count(j, _):
    for k in range(CH // 16):
      idx = dst_v[j, pl.ds(k * 16, 16)]
      plsc.addupdate_scatter(deg_v, [idx], ones)
    return 0
  lax.fori_loop(0, NCH_D, count, 0)

  pltpu.sync_copy(deg_v, out_hbm.at[tile])


def _deg_kernel(dst3):
  return pl.kernel(
      _deg_body,
      out_type=jax.ShapeDtypeStruct((NW, N_PAD), jnp.float32),
      mesh=_mesh,
      scratch_types=[
          pltpu.VMEM((NCH_D, CH), jnp.int32),
          pltpu.VMEM((N_PAD,), jnp.float32),
      ],
      compiler_params=_sc_params,
  )(dst3)


def _edge_body(h_hbm, idx_hbm, zeros_hbm, out_hbm,
               ibs, rws, h_sh, acc_sh, sgs, sis, *, dh):
  c = lax.axis_index("c")
  s = lax.axis_index("s")

  pltpu.sync_copy(h_hbm.at[c, pl.ds(s * STRIPE, STRIPE)],
                  h_sh.at[pl.ds(s * STRIPE, STRIPE)])
  pltpu.sync_copy(zeros_hbm.at[pl.ds(s * STRIPE, STRIPE)],
                  acc_sh.at[pl.ds(s * STRIPE, STRIPE)])
  plsc.subcore_barrier()

  for b in range(4):
    pltpu.async_copy(idx_hbm.at[s, b], ibs[b], sis[b])
  for b in range(3):
    pltpu.make_async_copy(idx_hbm.at[s, b], ibs[b], sis[b]).wait()
    pltpu.async_copy(h_sh.at[ibs[b].at[0]], rws[b], sgs[b])

  def group(g, _):
    for b in range(4):
      j = g * 4 + b
      b3 = (b + 3) % 4

      @pl.when(j + 3 < NCH_E)
      def _():
        pltpu.make_async_copy(idx_hbm.at[s, j + 3], ibs[b3], sis[b3]).wait()
        pltpu.async_copy(h_sh.at[ibs[b3].at[0]], rws[b3], sgs[b3])

      pltpu.make_async_copy(h_sh.at[ibs[b].at[0]], rws[b], sgs[b]).wait()
      pltpu.sync_copy(rws[b], acc_sh.at[ibs[b].at[1]], add=True)

      @pl.when(j + 4 < NCH_E)
      def _():
        pltpu.async_copy(idx_hbm.at[s, j + 4], ibs[b], sis[b])
    return 0

  lax.fori_loop(0, NCH_E // 4, group, 0)
  plsc.subcore_barrier()

  pltpu.sync_copy(acc_sh.at[pl.ds(s * STRIPE, STRIPE)],
                  out_hbm.at[c, pl.ds(s * STRIPE, STRIPE)])


def _edge_kernel(h, idx4, zeros_nd, dh):
  body = functools.partial(_edge_body, dh=dh)
  return pl.kernel(
      body,
      out_type=jax.ShapeDtypeStruct((NC, N_PAD, dh), jnp.float32),
      mesh=_mesh,
      scratch_types=[
          [pltpu.VMEM((2, CH), jnp.int32)] * 4,
          [pltpu.VMEM((CH, dh), jnp.float32)] * 4,
          pltpu.VMEM_SHARED((N_PAD, dh), jnp.float32),
          pltpu.VMEM_SHARED((N_PAD, dh), jnp.float32),
          [pltpu.SemaphoreType.DMA] * 4,
          [pltpu.SemaphoreType.DMA] * 4,
      ],
      compiler_params=_sc_params,
  )(h, idx4, zeros_nd)


_R = 2000


def _dis(degt_ref):
  deg = 1.0 + jnp.sum(degt_ref[...], axis=1, keepdims=True)
  return lax.rsqrt(deg)


def _scale_in_body(x_ref, w_ref, degt_ref, out_ref):
  dis = _dis(degt_ref)
  h = jnp.dot(x_ref[...], w_ref[0], preferred_element_type=jnp.float32)
  out_ref[0] = h * dis


def _tc_scale_in(x, w, degt, d_in, dh):
  return pl.pallas_call(
      _scale_in_body,
      grid=(N // _R, NC),
      in_specs=[
          pl.BlockSpec((_R, d_in), lambda j, c: (j, 0)),
          pl.BlockSpec((1, d_in, dh), lambda j, c: (c, 0, 0)),
          pl.BlockSpec((_R, NW), lambda j, c: (j, 0)),
      ],
      out_specs=pl.BlockSpec((1, _R, dh), lambda j, c: (c, j, 0)),
      out_shape=jax.ShapeDtypeStruct((NC, N_PAD, dh), jnp.float32),
  )(x, w, degt)


def _mid_body(acc_ref, hp_ref, degt_ref, b_ref, w_ref, out_ref):
  dis = _dis(degt_ref)
  tot = jnp.concatenate(
      [acc_ref[0] + hp_ref[0], acc_ref[1] + hp_ref[1]], axis=1)
  z = jnp.maximum(dis * tot + b_ref[...], 0.0)
  h2 = jnp.dot(z, w_ref[0], preferred_element_type=jnp.float32)
  out_ref[0] = h2 * dis


def _tc_mid(acc, hp, degt, b, w, dh_in, dh_out):
  return pl.pallas_call(
      _mid_body,
      grid=(N // _R, NC),
      in_specs=[
          pl.BlockSpec((NC, _R, dh_in), lambda j, c: (0, j, 0)),
          pl.BlockSpec((NC, _R, dh_in), lambda j, c: (0, j, 0)),
          pl.BlockSpec((_R, NW), lambda j, c: (j, 0)),
          pl.BlockSpec((1, 2 * dh_in), lambda j, c: (0, 0)),
          pl.BlockSpec((1, 2 * dh_in, dh_out), lambda j, c: (c, 0, 0)),
      ],
      out_specs=pl.BlockSpec((1, _R, dh_out), lambda j, c: (c, j, 0)),
      out_shape=jax.ShapeDtypeStruct((NC, N_PAD, dh_out), jnp.float32),
  )(acc, hp, degt, b, w)


def _final_body(acc_ref, hp_ref, degt_ref, b_ref, out_ref):
  dis = _dis(degt_ref)
  tot = jnp.concatenate(
      [acc_ref[0] + hp_ref[0], acc_ref[1] + hp_ref[1]], axis=1)
  out_ref[...] = jnp.maximum(dis * tot + b_ref[...], 0.0)


def _tc_final(acc, hp, degt, b, dh):
  return pl.pallas_call(
      _final_body,
      grid=(N // _R,),
      in_specs=[
          pl.BlockSpec((NC, _R, dh), lambda j: (0, j, 0)),
          pl.BlockSpec((NC, _R, dh), lambda j: (0, j, 0)),
          pl.BlockSpec((_R, NW), lambda j: (j, 0)),
          pl.BlockSpec((1, 2 * dh), lambda j: (0, 0)),
      ],
      out_specs=pl.BlockSpec((_R, 2 * dh), lambda j: (j, 0)),
      out_shape=jax.ShapeDtypeStruct((N, 2 * dh), jnp.float32),
  )(acc, hp, degt, b)


def kernel(x, edge_index, W1, b1, W2, b2):
  src = edge_index[0].astype(jnp.int32)
  dst = edge_index[1].astype(jnp.int32)
  pad = E_PAD - E
  src_p = jnp.concatenate([src, jnp.zeros((pad,), jnp.int32)])
  dst_p = jnp.concatenate([dst, jnp.full((pad,), DUMMY, jnp.int32)])
  dst3 = dst_p.reshape(NW, NCH_D, CH)
  idx4 = jnp.stack([src_p.reshape(NS, NCH_E, CH),
                    dst_p.reshape(NS, NCH_E, CH)], axis=2)

  deg_parts = _deg_kernel(dst3)
  degt = deg_parts.T[:N]

  zeros_64 = jnp.zeros((N_PAD, D_HID // 2), jnp.float32)
  zeros_32 = jnp.zeros((N_PAD, D_OUT // 2), jnp.float32)

  W1s = W1.reshape(D_IN, NC, D_HID // 2).transpose(1, 0, 2)
  W2s = W2.reshape(D_HID, NC, D_OUT // 2).transpose(1, 0, 2)

  h1p = _tc_scale_in(x, W1s, degt, D_IN, D_HID // 2)
  acc1 = _edge_kernel(h1p, idx4, zeros_64, D_HID // 2)
  h2p = _tc_mid(acc1, h1p, degt, b1.reshape(1, D_HID), W2s,
                D_HID // 2, D_OUT // 2)
  acc2 = _edge_kernel(h2p, idx4, zeros_32, D_OUT // 2)
  out = _tc_final(acc2, h2p, degt, b2.reshape(1, D_OUT), D_OUT // 2)
  return out
